# initial kernel scaffold (unmeasured)
import jax
import jax.numpy as jnp
from jax import lax
from jax.experimental import pallas as pl
from jax.experimental.pallas import tpu as pltpu


def kernel(
    x,
):
    def body(*refs):
        pass

    out_shape = jax.ShapeDtypeStruct(..., jnp.float32)
    return pl.pallas_call(body, out_shape=out_shape)(...)


# baseline (device time: 398778 ns/iter reference)
import jax
import jax.numpy as jnp
from jax import lax
from jax.experimental import pallas as pl
from jax.experimental.pallas import tpu as pltpu

NC = 16


def kernel(x):
    m, n = x.shape
    rows = m // NC

    def body(x_ref, out_ref, stage, send_sems, recv_sems, copy_sems):
        my_x = lax.axis_index("x")
        my_y = lax.axis_index("y")
        my_z = lax.axis_index("z")
        ynbr = (my_x, 1 - my_y, my_z)

        barrier_sem = pltpu.get_barrier_semaphore()
        pl.semaphore_signal(
            barrier_sem, inc=1, device_id=ynbr,
            device_id_type=pl.DeviceIdType.MESH,
        )
        pl.semaphore_wait(barrier_sem, 1)

        descs = []
        for k in range(NC):
            d = pltpu.make_async_remote_copy(
                src_ref=x_ref.at[pl.ds(k * rows, rows), :],
                dst_ref=out_ref.at[pl.ds(k * rows, rows), :],
                send_sem=send_sems.at[k],
                recv_sem=recv_sems.at[k],
                device_id=ynbr,
                device_id_type=pl.DeviceIdType.MESH,
            )
            d.start()
            descs.append(d)

        def local_copy(k):
            return pltpu.make_async_copy(
                x_ref.at[pl.ds(k * rows, rows), :],
                stage.at[k % 2],
                copy_sems.at[k % 2],
            )

        local_copy(0).start()
        for k in range(NC):
            if k + 1 < NC:
                local_copy(k + 1).start()
            descs[k].wait_recv()
            local_copy(k).wait()
            sl = pl.ds(k * rows, rows)
            out_ref[sl, :] = out_ref[sl, :] + stage[k % 2]

        for k in range(NC):
            descs[k].wait_send()

    return pl.pallas_call(
        body,
        out_shape=jax.ShapeDtypeStruct((m, n), x.dtype),
        in_specs=[pl.BlockSpec(memory_space=pl.ANY)],
        out_specs=pl.BlockSpec(memory_space=pltpu.VMEM),
        scratch_shapes=[
            pltpu.VMEM((2, rows, n), x.dtype),
            pltpu.SemaphoreType.DMA((NC,)),
            pltpu.SemaphoreType.DMA((NC,)),
            pltpu.SemaphoreType.DMA((2,)),
        ],
        compiler_params=pltpu.CompilerParams(
            collective_id=0,
            vmem_limit_bytes=60 * 1024 * 1024,
        ),
    )(x)


# device time: 244107 ns/iter; 1.6336x vs baseline; 1.6336x over previous
import jax
import jax.numpy as jnp
from jax import lax
from jax.experimental import pallas as pl
from jax.experimental.pallas import tpu as pltpu

NC = 16


def kernel(x):
    m, n = x.shape
    half = m // 2
    rows = half // NC

    def body(
        x_ref, out_ref, stage,
        ysend_sems, yrecv_sems, fsend_sems, frecv_sems, copy_sems,
    ):
        my_x = lax.axis_index("x")
        my_y = lax.axis_index("y")
        my_z = lax.axis_index("z")
        ynbr = (my_x, 1 - my_y, my_z)
        xnbr = (1 - my_x, my_y, my_z)
        dir_base = my_x * half
        fwd_base = (1 - my_x) * half

        barrier_sem = pltpu.get_barrier_semaphore()
        for nbr in (ynbr, xnbr):
            pl.semaphore_signal(
                barrier_sem, inc=1, device_id=nbr,
                device_id_type=pl.DeviceIdType.MESH,
            )
        pl.semaphore_wait(barrier_sem, 2)

        ydescs = []
        for k in range(NC):
            sl = pl.ds(dir_base + k * rows, rows)
            d = pltpu.make_async_remote_copy(
                src_ref=x_ref.at[sl, :],
                dst_ref=out_ref.at[sl, :],
                send_sem=ysend_sems.at[k],
                recv_sem=yrecv_sems.at[k],
                device_id=ynbr,
                device_id_type=pl.DeviceIdType.MESH,
            )
            d.start()
            ydescs.append(d)

        fdescs = []
        for k in range(NC):
            sl = pl.ds(dir_base + k * rows, rows)
            fdescs.append(
                pltpu.make_async_remote_copy(
                    src_ref=out_ref.at[sl, :],
                    dst_ref=out_ref.at[sl, :],
                    send_sem=fsend_sems.at[k],
                    recv_sem=frecv_sems.at[k],
                    device_id=xnbr,
                    device_id_type=pl.DeviceIdType.MESH,
                )
            )

        for k in range(NC):
            ydescs[k].wait_recv()
            fdescs[k].start()

        def add_target(j):
            if j < NC:
                return fwd_base + j * rows
            return dir_base + (j - NC) * rows

        def stage_copy(j, slot):
            return pltpu.make_async_copy(
                x_ref.at[pl.ds(add_target(j), rows), :],
                stage.at[slot],
                copy_sems.at[slot],
            )

        stage_copy(0, 0).start()
        for j in range(2 * NC):
            if j + 1 < 2 * NC:
                stage_copy(j + 1, (j + 1) % 2).start()
            if j < NC:
                fdescs[j].wait_recv()
            else:
                fdescs[j - NC].wait_send()
            stage_copy(j, j % 2).wait()
            sl = pl.ds(add_target(j), rows)
            out_ref[sl, :] = out_ref[sl, :] + stage[j % 2]

        for k in range(NC):
            ydescs[k].wait_send()

    return pl.pallas_call(
        body,
        out_shape=jax.ShapeDtypeStruct((m, n), x.dtype),
        in_specs=[pl.BlockSpec(memory_space=pl.ANY)],
        out_specs=pl.BlockSpec(memory_space=pltpu.VMEM),
        scratch_shapes=[
            pltpu.VMEM((2, rows, n), x.dtype),
            pltpu.SemaphoreType.DMA((NC,)),
            pltpu.SemaphoreType.DMA((NC,)),
            pltpu.SemaphoreType.DMA((NC,)),
            pltpu.SemaphoreType.DMA((NC,)),
            pltpu.SemaphoreType.DMA((2,)),
        ],
        compiler_params=pltpu.CompilerParams(
            collective_id=0,
            vmem_limit_bytes=60 * 1024 * 1024,
        ),
    )(x)
